# baseline (speedup 1.0000x reference)
import jax
import jax.numpy as jnp
from jax import lax
from jax.experimental import pallas as pl
from jax.experimental.pallas import tpu as pltpu
from jax.experimental.pallas import tpu_sc as plsc

_C = 8
_W = 256
_B = 512
_NW = 32
_L = 128

_SC_PARAMS = pltpu.CompilerParams(needs_layout_passes=False,
                                  use_tc_tiling_on_sc=True)


def _route_body(t_ref, rank_ref, ps_ref, be_ref):
    t = t_ref[...]
    rows = t.shape[0]
    nb = be_ref.shape[0]
    rank = jnp.zeros_like(t)
    totals = []
    for c in range(_C):
        oh = (t == c).astype(jnp.int32)
        pre = oh
        s = 1
        while s < _L:
            pre = pre + jnp.concatenate(
                [jnp.zeros((rows, s), jnp.int32), pre[:, : _L - s]], axis=1)
            s *= 2
        row_tot = pre[:, _L - 1 :]
        inc = row_tot
        s = 1
        while s < rows:
            inc = inc + jnp.concatenate(
                [jnp.zeros((s, 1), jnp.int32), inc[: rows - s, :]], axis=0)
            s *= 2
        row_off = inc - row_tot
        pre_full = pre + row_off
        rank = rank + jnp.where(oh == 1, pre_full - 1, 0)
        totals.append(inc[rows - 1 :, :])
    rank_ref[...] = rank

    run = jnp.zeros((1, 1), jnp.int32)
    ps_pieces, cum_pieces = [], []
    for c in range(_C):
        nblk_c = (totals[c] + (_B - 1)) // _B
        ps_pieces.append(run * _B)
        run = run + nblk_c
        cum_pieces.append(run)
    ps_col = jnp.concatenate(ps_pieces + [jnp.zeros((8, 1), jnp.int32)],
                             axis=0)
    ps_ref[...] = ps_col
    cum_row = jnp.concatenate(cum_pieces, axis=1)
    bid = lax.broadcasted_iota(jnp.int32, (nb, _C), 0)
    be = jnp.minimum(
        jnp.sum((bid >= cum_row).astype(jnp.int32), axis=1, keepdims=True),
        _C - 1)
    be_ref[...] = be


def _route(times2d, nb):
    return pl.pallas_call(
        _route_body,
        out_shape=(
            jax.ShapeDtypeStruct(times2d.shape, jnp.int32),
            jax.ShapeDtypeStruct((16, 1), jnp.int32),
            jax.ShapeDtypeStruct((nb, 1), jnp.int32),
        ),
    )(times2d)


def _dispatch_body(posp_hbm, times2_hbm, rank2_hbm, ps_hbm,
                   xpad_hbm, slot_hbm,
                   t2_v, r2_v, ps_v, idx2_v, pos_v, sem):
    wid = lax.axis_index("s") * 2 + lax.axis_index("c")
    rb = wid * 8
    base = wid * 1024
    pltpu.sync_copy(times2_hbm.at[pl.ds(rb, 8)], t2_v)
    pltpu.sync_copy(rank2_hbm.at[pl.ds(rb, 8)], r2_v)
    pltpu.sync_copy(ps_hbm, ps_v)

    for row in range(8):
        def body(k, carry, row=row):
            t = t2_v[row, pl.ds(k * 16, 16)]
            r = r2_v[row, pl.ds(k * 16, 16)]
            ps = plsc.load_gather(ps_v, [t])
            idx2_v[row, pl.ds(k * 16, 16)] = ps + r
            return carry
        lax.fori_loop(0, 8, body, 0)

    for row in range(8):
        pltpu.sync_copy(idx2_v.at[row],
                        slot_hbm.at[pl.ds(base + row * 128, 128)])

    for half in range(2):
        pltpu.sync_copy(posp_hbm.at[pl.ds(base + half * 512, 512)], pos_v)
        cps = [
            pltpu.async_copy(pos_v.at[pl.ds(j * 128, 128)],
                             xpad_hbm.at[idx2_v.at[half * 4 + j]], sem)
            for j in range(4)
        ]
        for cp in cps:
            cp.wait()


def _dispatch(posp, times2, rank2, ps16, npad):
    n = posp.shape[0]
    mesh = plsc.VectorSubcoreMesh(core_axis_name="c", subcore_axis_name="s")
    return pl.kernel(
        _dispatch_body,
        out_type=(
            jax.ShapeDtypeStruct((npad, _L), jnp.float32),
            jax.ShapeDtypeStruct((n,), jnp.int32),
        ),
        mesh=mesh,
        scratch_types=[
            pltpu.VMEM((8, _L), jnp.int32),
            pltpu.VMEM((8, _L), jnp.int32),
            pltpu.VMEM((16,), jnp.int32),
            pltpu.VMEM((8, _L), jnp.int32),
            pltpu.VMEM((512, _L), jnp.float32),
            pltpu.SemaphoreType.DMA,
        ],
        compiler_params=_SC_PARAMS,
    )(posp, times2, rank2, ps16)


def _mlp_body(e_ref, x_ref, w1_ref, b1_ref, w2_ref, b2_ref,
              w3_ref, b3_ref, w4_ref, b4_ref, y_ref):
    x = x_ref[...]
    w1p = jnp.concatenate(
        [w1_ref[0], jnp.zeros((_L - 3, _W), jnp.float32)], axis=0)
    h = jnp.tanh(jnp.dot(x.astype(jnp.bfloat16), w1p.astype(jnp.bfloat16),
                         preferred_element_type=jnp.float32) + b1_ref[0])
    h = jnp.tanh(jnp.dot(h.astype(jnp.bfloat16),
                         w2_ref[0].astype(jnp.bfloat16),
                         preferred_element_type=jnp.float32) + b2_ref[0])
    h = jnp.tanh(jnp.dot(h.astype(jnp.bfloat16),
                         w3_ref[0].astype(jnp.bfloat16),
                         preferred_element_type=jnp.float32) + b3_ref[0])
    y = jnp.tanh(jnp.dot(h.astype(jnp.bfloat16),
                         w4_ref[0].astype(jnp.bfloat16),
                         preferred_element_type=jnp.float32)
                 + b4_ref[0])
    y_ref[...] = jnp.concatenate(
        [y, jnp.zeros((y.shape[0], _L - 3), jnp.float32)], axis=1)


def _mlp(block_expert, xpad, w1, b1, w2, b2, w3, b3, w4, b4):
    npad = xpad.shape[0]
    nb = npad // _B
    grid_spec = pltpu.PrefetchScalarGridSpec(
        num_scalar_prefetch=1,
        grid=(nb,),
        in_specs=[
            pl.BlockSpec((_B, _L), lambda i, e: (i, 0)),
            pl.BlockSpec((1, 3, _W), lambda i, e: (e[i, 0], 0, 0)),
            pl.BlockSpec((1, 1, _W), lambda i, e: (e[i, 0], 0, 0)),
            pl.BlockSpec((1, _W, _W), lambda i, e: (e[i, 0], 0, 0)),
            pl.BlockSpec((1, 1, _W), lambda i, e: (e[i, 0], 0, 0)),
            pl.BlockSpec((1, _W, _W), lambda i, e: (e[i, 0], 0, 0)),
            pl.BlockSpec((1, 1, _W), lambda i, e: (e[i, 0], 0, 0)),
            pl.BlockSpec((1, _W, 3), lambda i, e: (e[i, 0], 0, 0)),
            pl.BlockSpec((1, 1, 3), lambda i, e: (e[i, 0], 0, 0)),
        ],
        out_specs=pl.BlockSpec((_B, _L), lambda i, e: (i, 0)),
    )
    return pl.pallas_call(
        _mlp_body,
        grid_spec=grid_spec,
        out_shape=jax.ShapeDtypeStruct((npad, _L), jnp.float32),
    )(block_expert, xpad, w1,
      b1.reshape(_C, 1, _W), w2, b2.reshape(_C, 1, _W),
      w3, b3.reshape(_C, 1, _W), w4, b4.reshape(_C, 1, 3))


def _combine_body(ypad_hbm, slot_hbm, out_hbm, idx1_v, y_v, sem):
    wid = lax.axis_index("s") * 2 + lax.axis_index("c")
    base = wid * 1024
    pltpu.sync_copy(slot_hbm.at[pl.ds(base, 1024)], idx1_v)
    for half in range(2):
        cps = [
            pltpu.async_copy(
                ypad_hbm.at[idx1_v.at[pl.ds(half * 512 + j * 128, 128)]],
                y_v.at[pl.ds(j * 128, 128)], sem)
            for j in range(4)
        ]
        for cp in cps:
            cp.wait()
        pltpu.sync_copy(y_v, out_hbm.at[pl.ds(base + half * 512, 512)])


def _combine(ypad, slot):
    n = slot.shape[0]
    mesh = plsc.VectorSubcoreMesh(core_axis_name="c", subcore_axis_name="s")
    return pl.kernel(
        _combine_body,
        out_type=jax.ShapeDtypeStruct((n, _L), jnp.float32),
        mesh=mesh,
        scratch_types=[
            pltpu.VMEM((1024,), jnp.int32),
            pltpu.VMEM((512, _L), jnp.float32),
            pltpu.SemaphoreType.DMA,
        ],
        compiler_params=_SC_PARAMS,
    )(ypad, slot)


def kernel(positions, times, W1, b1, W2, b2, W3, b3, W4, b4):
    n = positions.shape[0]
    npad = n + _C * _B
    nb = npad // _B

    times2 = times.astype(jnp.int32).reshape(n // _L, _L)
    rank2, ps2d, block_expert = _route(times2, nb)
    posp = jnp.pad(positions, ((0, 0), (0, _L - 3)))
    xpad, slot = _dispatch(posp, times2, rank2, ps2d.reshape(16), npad)
    ypad = _mlp(block_expert, xpad, W1, b1, W2, b2, W3, b3, W4, b4)
    out = _combine(ypad, slot)
    return out[:, :3]


# speedup vs baseline: 1.1586x; 1.1586x over previous
import jax
import jax.numpy as jnp
from jax import lax
from jax.experimental import pallas as pl
from jax.experimental.pallas import tpu as pltpu
from jax.experimental.pallas import tpu_sc as plsc

_C = 8
_W = 256
_B = 1024
_NW = 32
_L = 128

_SC_PARAMS = pltpu.CompilerParams(needs_layout_passes=False,
                                  use_tc_tiling_on_sc=True)


def _route_body(t_ref, rank_ref, ps_ref, be_ref):
    t = t_ref[...]
    rows = t.shape[0]
    nb = be_ref.shape[0]
    rank = jnp.zeros_like(t)
    totals = []
    for c in range(_C):
        oh = (t == c).astype(jnp.int32)
        pre = oh
        s = 1
        while s < _L:
            pre = pre + jnp.concatenate(
                [jnp.zeros((rows, s), jnp.int32), pre[:, : _L - s]], axis=1)
            s *= 2
        row_tot = pre[:, _L - 1 :]
        inc = row_tot
        s = 1
        while s < rows:
            inc = inc + jnp.concatenate(
                [jnp.zeros((s, 1), jnp.int32), inc[: rows - s, :]], axis=0)
            s *= 2
        row_off = inc - row_tot
        pre_full = pre + row_off
        rank = rank + jnp.where(oh == 1, pre_full - 1, 0)
        totals.append(inc[rows - 1 :, :])
    rank_ref[...] = rank

    run = jnp.zeros((1, 1), jnp.int32)
    ps_pieces, cum_pieces = [], []
    for c in range(_C):
        nblk_c = (totals[c] + (_B - 1)) // _B
        ps_pieces.append(run * _B)
        run = run + nblk_c
        cum_pieces.append(run)
    ps_col = jnp.concatenate(ps_pieces + [jnp.zeros((8, 1), jnp.int32)],
                             axis=0)
    ps_ref[...] = ps_col
    cum_row = jnp.concatenate(cum_pieces, axis=1)
    bid = lax.broadcasted_iota(jnp.int32, (nb, _C), 0)
    be = jnp.minimum(
        jnp.sum((bid >= cum_row).astype(jnp.int32), axis=1, keepdims=True),
        _C - 1)
    be_ref[...] = be


def _route(times2d, nb):
    return pl.pallas_call(
        _route_body,
        out_shape=(
            jax.ShapeDtypeStruct(times2d.shape, jnp.int32),
            jax.ShapeDtypeStruct((16, 1), jnp.int32),
            jax.ShapeDtypeStruct((nb, 1), jnp.int32),
        ),
    )(times2d)


def _dispatch_body(posp_hbm, times2_hbm, rank2_hbm, ps_hbm,
                   xpad_hbm, slot_hbm,
                   t2_v, r2_v, ps_v, idx2_v, pos_v, sem):
    wid = lax.axis_index("s") * 2 + lax.axis_index("c")
    rb = wid * 8
    base = wid * 1024
    pltpu.sync_copy(times2_hbm.at[pl.ds(rb, 8)], t2_v)
    pltpu.sync_copy(rank2_hbm.at[pl.ds(rb, 8)], r2_v)
    pltpu.sync_copy(ps_hbm, ps_v)

    for row in range(8):
        def body(k, carry, row=row):
            t = t2_v[row, pl.ds(k * 16, 16)]
            r = r2_v[row, pl.ds(k * 16, 16)]
            ps = plsc.load_gather(ps_v, [t])
            idx2_v[row, pl.ds(k * 16, 16)] = ps + r
            return carry
        lax.fori_loop(0, 8, body, 0)

    for row in range(8):
        pltpu.sync_copy(idx2_v.at[row],
                        slot_hbm.at[pl.ds(base + row * 128, 128)])

    for half in range(2):
        pltpu.sync_copy(posp_hbm.at[pl.ds(base + half * 512, 512)], pos_v)
        cps = [
            pltpu.async_copy(pos_v.at[pl.ds(j * 128, 128)],
                             xpad_hbm.at[idx2_v.at[half * 4 + j]], sem)
            for j in range(4)
        ]
        for cp in cps:
            cp.wait()


def _dispatch(posp, times2, rank2, ps16, npad):
    n = posp.shape[0]
    mesh = plsc.VectorSubcoreMesh(core_axis_name="c", subcore_axis_name="s")
    return pl.kernel(
        _dispatch_body,
        out_type=(
            jax.ShapeDtypeStruct((npad, _L), jnp.float32),
            jax.ShapeDtypeStruct((n,), jnp.int32),
        ),
        mesh=mesh,
        scratch_types=[
            pltpu.VMEM((8, _L), jnp.int32),
            pltpu.VMEM((8, _L), jnp.int32),
            pltpu.VMEM((16,), jnp.int32),
            pltpu.VMEM((8, _L), jnp.int32),
            pltpu.VMEM((512, _L), jnp.float32),
            pltpu.SemaphoreType.DMA,
        ],
        compiler_params=_SC_PARAMS,
    )(posp, times2, rank2, ps16)


def _mlp_body(e_ref, x_ref, w1_ref, b1_ref, w2_ref, b2_ref,
              w3_ref, b3_ref, w4_ref, b4_ref, y_ref):
    x = x_ref[...]
    w1p = jnp.concatenate(
        [w1_ref[0], jnp.zeros((_L - 3, _W), jnp.float32)],
        axis=0).astype(jnp.bfloat16)
    w2 = w2_ref[0].astype(jnp.bfloat16)
    w3 = w3_ref[0].astype(jnp.bfloat16)
    w4 = w4_ref[0].astype(jnp.bfloat16)
    b = x.shape[0]
    h = jnp.tanh(jnp.dot(x.astype(jnp.bfloat16), w1p,
                         preferred_element_type=jnp.float32) + b1_ref[0])
    h = jnp.tanh(jnp.dot(h.astype(jnp.bfloat16), w2,
                         preferred_element_type=jnp.float32) + b2_ref[0])
    h = jnp.tanh(jnp.dot(h.astype(jnp.bfloat16), w3,
                         preferred_element_type=jnp.float32) + b3_ref[0])
    y = jnp.tanh(jnp.dot(h.astype(jnp.bfloat16), w4,
                         preferred_element_type=jnp.float32) + b4_ref[0])
    y_ref[...] = jnp.concatenate(
        [y, jnp.zeros((b, _L - 3), jnp.float32)], axis=1)


def _mlp(block_expert, xpad, w1, b1, w2, b2, w3, b3, w4, b4):
    npad = xpad.shape[0]
    nb = npad // _B
    grid_spec = pltpu.PrefetchScalarGridSpec(
        num_scalar_prefetch=1,
        grid=(nb,),
        in_specs=[
            pl.BlockSpec((_B, _L), lambda i, e: (i, 0)),
            pl.BlockSpec((1, 3, _W), lambda i, e: (e[i, 0], 0, 0)),
            pl.BlockSpec((1, 1, _W), lambda i, e: (e[i, 0], 0, 0)),
            pl.BlockSpec((1, _W, _W), lambda i, e: (e[i, 0], 0, 0)),
            pl.BlockSpec((1, 1, _W), lambda i, e: (e[i, 0], 0, 0)),
            pl.BlockSpec((1, _W, _W), lambda i, e: (e[i, 0], 0, 0)),
            pl.BlockSpec((1, 1, _W), lambda i, e: (e[i, 0], 0, 0)),
            pl.BlockSpec((1, _W, 3), lambda i, e: (e[i, 0], 0, 0)),
            pl.BlockSpec((1, 1, 3), lambda i, e: (e[i, 0], 0, 0)),
        ],
        out_specs=pl.BlockSpec((_B, _L), lambda i, e: (i, 0)),
    )
    return pl.pallas_call(
        _mlp_body,
        grid_spec=grid_spec,
        out_shape=jax.ShapeDtypeStruct((npad, _L), jnp.float32),
    )(block_expert, xpad, w1,
      b1.reshape(_C, 1, _W), w2, b2.reshape(_C, 1, _W),
      w3, b3.reshape(_C, 1, _W), w4, b4.reshape(_C, 1, 3))


def _combine_body(ypad_hbm, slot_hbm, out_hbm, idx1_v, y_v, sem):
    wid = lax.axis_index("s") * 2 + lax.axis_index("c")
    base = wid * 1024
    pltpu.sync_copy(slot_hbm.at[pl.ds(base, 1024)], idx1_v)
    for half in range(2):
        cps = [
            pltpu.async_copy(
                ypad_hbm.at[idx1_v.at[pl.ds(half * 512 + j * 128, 128)]],
                y_v.at[pl.ds(j * 128, 128)], sem)
            for j in range(4)
        ]
        for cp in cps:
            cp.wait()
        pltpu.sync_copy(y_v, out_hbm.at[pl.ds(base + half * 512, 512)])


def _combine(ypad, slot):
    n = slot.shape[0]
    mesh = plsc.VectorSubcoreMesh(core_axis_name="c", subcore_axis_name="s")
    return pl.kernel(
        _combine_body,
        out_type=jax.ShapeDtypeStruct((n, _L), jnp.float32),
        mesh=mesh,
        scratch_types=[
            pltpu.VMEM((1024,), jnp.int32),
            pltpu.VMEM((512, _L), jnp.float32),
            pltpu.SemaphoreType.DMA,
        ],
        compiler_params=_SC_PARAMS,
    )(ypad, slot)


def kernel(positions, times, W1, b1, W2, b2, W3, b3, W4, b4):
    n = positions.shape[0]
    npad = n + _C * _B
    nb = npad // _B

    times2 = times.astype(jnp.int32).reshape(n // _L, _L)
    rank2, ps2d, block_expert = _route(times2, nb)
    posp = jnp.pad(positions, ((0, 0), (0, _L - 3)))
    xpad, slot = _dispatch(posp, times2, rank2, ps2d.reshape(16), npad)
    ypad = _mlp(block_expert, xpad, W1, b1, W2, b2, W3, b3, W4, b4)
    out = _combine(ypad, slot)
    return out[:, :3]


# speedup vs baseline: 1.4300x; 1.2342x over previous
import jax
import jax.numpy as jnp
from jax import lax
from jax.experimental import pallas as pl
from jax.experimental.pallas import tpu as pltpu
from jax.experimental.pallas import tpu_sc as plsc

_C = 8
_W = 256
_B = 1024
_NW = 32
_L = 128

_SC_PARAMS = pltpu.CompilerParams(needs_layout_passes=False,
                                  use_tc_tiling_on_sc=True)


def _route_body(t_ref, slot_ref, be_ref):
    t = t_ref[...]
    rows = t.shape[0]
    nb = be_ref.shape[0]
    rank = jnp.zeros_like(t)
    ohs, totals = [], []
    for c in range(_C):
        oh = (t == c).astype(jnp.int32)
        ohs.append(oh)
        pre = oh
        s = 1
        while s < _L:
            pre = pre + jnp.concatenate(
                [jnp.zeros((rows, s), jnp.int32), pre[:, : _L - s]], axis=1)
            s *= 2
        row_tot = pre[:, _L - 1 :]
        inc = row_tot
        s = 1
        while s < rows:
            inc = inc + jnp.concatenate(
                [jnp.zeros((s, 1), jnp.int32), inc[: rows - s, :]], axis=0)
            s *= 2
        row_off = inc - row_tot
        pre_full = pre + row_off
        rank = rank + jnp.where(oh == 1, pre_full - 1, 0)
        totals.append(inc[rows - 1 :, :])

    run = jnp.zeros((1, 1), jnp.int32)
    slot = rank
    cum_pieces = []
    for c in range(_C):
        nblk_c = (totals[c] + (_B - 1)) // _B
        slot = slot + ohs[c] * (run * _B)
        run = run + nblk_c
        cum_pieces.append(run)
    slot_ref[...] = slot
    cum_row = jnp.concatenate(cum_pieces, axis=1)
    bid = lax.broadcasted_iota(jnp.int32, (nb, _C), 0)
    be = jnp.minimum(
        jnp.sum((bid >= cum_row).astype(jnp.int32), axis=1, keepdims=True),
        _C - 1)
    be_ref[...] = be


def _route(times2d, nb):
    return pl.pallas_call(
        _route_body,
        out_shape=(
            jax.ShapeDtypeStruct(times2d.shape, jnp.int32),
            jax.ShapeDtypeStruct((nb, 1), jnp.int32),
        ),
    )(times2d)


def _dispatch_body(post_hbm, slot2_hbm, xpad_hbm,
                   idx2_v, post_v, pos_v, sem):
    wid = lax.axis_index("s") * 2 + lax.axis_index("c")
    base = wid * 1024
    pltpu.sync_copy(slot2_hbm.at[pl.ds(wid * 8, 8)], idx2_v)
    pltpu.sync_copy(post_hbm.at[:, pl.ds(base, 1024)], post_v)

    lanes = lax.broadcasted_iota(jnp.int32, (16,), 0)
    for half in range(2):
        def body(j, carry, half=half):
            src = half * 512 + j * 16
            rows = lanes + j * 16
            for c in range(3):
                xv = post_v[c, pl.ds(src, 16)]
                plsc.store_scatter(pos_v, [rows, jnp.full((16,), c, jnp.int32)], xv)
            return carry
        lax.fori_loop(0, 32, body, 0)
        cps = [
            pltpu.async_copy(pos_v.at[pl.ds(j * 128, 128)],
                             xpad_hbm.at[idx2_v.at[half * 4 + j]], sem)
            for j in range(4)
        ]
        for cp in cps:
            cp.wait()


def _dispatch(post, slot2, npad):
    mesh = plsc.VectorSubcoreMesh(core_axis_name="c", subcore_axis_name="s")
    return pl.kernel(
        _dispatch_body,
        out_type=jax.ShapeDtypeStruct((npad, _L), jnp.float32),
        mesh=mesh,
        scratch_types=[
            pltpu.VMEM((8, _L), jnp.int32),
            pltpu.VMEM((8, 1024), jnp.float32),
            pltpu.VMEM((512, _L), jnp.float32),
            pltpu.SemaphoreType.DMA,
        ],
        compiler_params=_SC_PARAMS,
    )(post, slot2)


def _mlp_body(e_ref, x_ref, w1_ref, b1_ref, w2_ref, b2_ref,
              w3_ref, b3_ref, w4_ref, b4_ref, y_ref):
    x = x_ref[...][:, :3]
    b = x.shape[0]
    h = jnp.tanh(jnp.dot(x.astype(jnp.bfloat16),
                         w1_ref[0].astype(jnp.bfloat16),
                         preferred_element_type=jnp.float32) + b1_ref[0])
    h = jnp.tanh(jnp.dot(h.astype(jnp.bfloat16),
                         w2_ref[0].astype(jnp.bfloat16),
                         preferred_element_type=jnp.float32) + b2_ref[0])
    h = jnp.tanh(jnp.dot(h.astype(jnp.bfloat16),
                         w3_ref[0].astype(jnp.bfloat16),
                         preferred_element_type=jnp.float32) + b3_ref[0])
    y = jnp.tanh(jnp.dot(h.astype(jnp.bfloat16),
                         w4_ref[0].astype(jnp.bfloat16),
                         preferred_element_type=jnp.float32) + b4_ref[0])
    y_ref[...] = jnp.concatenate(
        [y, jnp.zeros((b, _L - 3), jnp.float32)], axis=1)


def _mlp(block_expert, xpad, w1, b1, w2, b2, w3, b3, w4, b4):
    npad = xpad.shape[0]
    nb = npad // _B
    grid_spec = pltpu.PrefetchScalarGridSpec(
        num_scalar_prefetch=1,
        grid=(nb,),
        in_specs=[
            pl.BlockSpec((_B, _L), lambda i, e: (i, 0)),
            pl.BlockSpec((1, 3, _W), lambda i, e: (e[i, 0], 0, 0)),
            pl.BlockSpec((1, 1, _W), lambda i, e: (e[i, 0], 0, 0)),
            pl.BlockSpec((1, _W, _W), lambda i, e: (e[i, 0], 0, 0)),
            pl.BlockSpec((1, 1, _W), lambda i, e: (e[i, 0], 0, 0)),
            pl.BlockSpec((1, _W, _W), lambda i, e: (e[i, 0], 0, 0)),
            pl.BlockSpec((1, 1, _W), lambda i, e: (e[i, 0], 0, 0)),
            pl.BlockSpec((1, _W, 3), lambda i, e: (e[i, 0], 0, 0)),
            pl.BlockSpec((1, 1, 3), lambda i, e: (e[i, 0], 0, 0)),
        ],
        out_specs=pl.BlockSpec((_B, _L), lambda i, e: (i, 0)),
    )
    return pl.pallas_call(
        _mlp_body,
        grid_spec=grid_spec,
        out_shape=jax.ShapeDtypeStruct((npad, _L), jnp.float32),
    )(block_expert, xpad, w1,
      b1.reshape(_C, 1, _W), w2, b2.reshape(_C, 1, _W),
      w3, b3.reshape(_C, 1, _W), w4, b4.reshape(_C, 1, 3))


def _combine_body(ypad_hbm, slot2_hbm, out_hbm, idx2_v, y_v, sem):
    wid = lax.axis_index("s") * 2 + lax.axis_index("c")
    base = wid * 1024
    pltpu.sync_copy(slot2_hbm.at[pl.ds(wid * 8, 8)], idx2_v)
    for half in range(2):
        cps = [
            pltpu.async_copy(ypad_hbm.at[idx2_v.at[half * 4 + j]],
                             y_v.at[pl.ds(j * 128, 128)], sem)
            for j in range(4)
        ]
        for cp in cps:
            cp.wait()
        pltpu.sync_copy(y_v, out_hbm.at[pl.ds(base + half * 512, 512)])


def _combine(ypad, slot2):
    n = slot2.shape[0] * slot2.shape[1]
    mesh = plsc.VectorSubcoreMesh(core_axis_name="c", subcore_axis_name="s")
    return pl.kernel(
        _combine_body,
        out_type=jax.ShapeDtypeStruct((n, _L), jnp.float32),
        mesh=mesh,
        scratch_types=[
            pltpu.VMEM((8, _L), jnp.int32),
            pltpu.VMEM((512, _L), jnp.float32),
            pltpu.SemaphoreType.DMA,
        ],
        compiler_params=_SC_PARAMS,
    )(ypad, slot2)


def kernel(positions, times, W1, b1, W2, b2, W3, b3, W4, b4):
    n = positions.shape[0]
    npad = n + _C * _B
    nb = npad // _B

    times2 = times.astype(jnp.int32).reshape(n // _L, _L)
    slot2, block_expert = _route(times2, nb)
    post = jnp.pad(positions.T, ((0, 5), (0, 0)))
    xpad = _dispatch(post, slot2, npad)
    ypad = _mlp(block_expert, xpad, W1, b1, W2, b2, W3, b3, W4, b4)
    out = _combine(ypad, slot2)
    return out[:, :3]


# speedup vs baseline: 1.5183x; 1.0617x over previous
import jax
import jax.numpy as jnp
from jax import lax
from jax.experimental import pallas as pl
from jax.experimental.pallas import tpu as pltpu
from jax.experimental.pallas import tpu_sc as plsc

_C = 8
_W = 256
_B = 1024
_NW = 32
_L = 128

_SC_PARAMS = pltpu.CompilerParams(needs_layout_passes=False,
                                  use_tc_tiling_on_sc=True)


def _route_body(t_ref, slot_ref, be_ref):
    t = t_ref[...]
    rows = t.shape[0]
    nb = be_ref.shape[0]
    rank = jnp.zeros_like(t)
    ohs, totals = [None] * _C, [None] * _C
    for p in range(_C // 2):
        oh_lo = (t == p).astype(jnp.int32)
        oh_hi = (t == p + _C // 2).astype(jnp.int32)
        ohs[p], ohs[p + _C // 2] = oh_lo, oh_hi
        packed = oh_lo + (oh_hi << 16)
        pre = packed
        s = 1
        while s < _L:
            pre = pre + jnp.concatenate(
                [jnp.zeros((rows, s), jnp.int32), pre[:, : _L - s]], axis=1)
            s *= 2
        row_tot = pre[:, _L - 1 :]
        inc = row_tot
        s = 1
        while s < rows:
            inc = inc + jnp.concatenate(
                [jnp.zeros((s, 1), jnp.int32), inc[: rows - s, :]], axis=0)
            s *= 2
        row_off = inc - row_tot
        pre_full = pre + row_off
        lo = pre_full & 0xFFFF
        hi = (pre_full >> 16) & 0xFFFF
        rank = (rank + jnp.where(oh_lo == 1, lo - 1, 0)
                + jnp.where(oh_hi == 1, hi - 1, 0))
        tot = inc[rows - 1 :, :]
        totals[p] = tot & 0xFFFF
        totals[p + _C // 2] = (tot >> 16) & 0xFFFF

    run = jnp.zeros((1, 1), jnp.int32)
    slot = rank
    cum_pieces = []
    for c in range(_C):
        nblk_c = (totals[c] + (_B - 1)) // _B
        slot = slot + ohs[c] * (run * _B)
        run = run + nblk_c
        cum_pieces.append(run)
    slot_ref[...] = slot
    cum_row = jnp.concatenate(cum_pieces, axis=1)
    bid = lax.broadcasted_iota(jnp.int32, (nb, _C), 0)
    be = jnp.minimum(
        jnp.sum((bid >= cum_row).astype(jnp.int32), axis=1, keepdims=True),
        _C - 1)
    be_ref[...] = be


def _route(times2d, nb):
    return pl.pallas_call(
        _route_body,
        out_shape=(
            jax.ShapeDtypeStruct(times2d.shape, jnp.int32),
            jax.ShapeDtypeStruct((nb, 1), jnp.int32),
        ),
    )(times2d)


def _dispatch_body(post_hbm, slot2_hbm, xpad_hbm,
                   idx2_v, post_v, pos_v, sem):
    wid = lax.axis_index("s") * 2 + lax.axis_index("c")
    base = wid * 1024
    pltpu.sync_copy(slot2_hbm.at[pl.ds(wid * 8, 8)], idx2_v)
    pltpu.sync_copy(post_hbm.at[:, pl.ds(base, 1024)], post_v)

    lanes = lax.broadcasted_iota(jnp.int32, (16,), 0)
    for half in range(2):
        def body(j, carry, half=half):
            src = half * 512 + j * 16
            rows = lanes + j * 16
            for c in range(3):
                xv = post_v[c, pl.ds(src, 16)]
                plsc.store_scatter(pos_v, [rows, jnp.full((16,), c, jnp.int32)], xv)
            return carry
        lax.fori_loop(0, 32, body, 0)
        cps = [
            pltpu.async_copy(pos_v.at[pl.ds(j * 128, 128)],
                             xpad_hbm.at[idx2_v.at[half * 4 + j]], sem)
            for j in range(4)
        ]
        for cp in cps:
            cp.wait()


def _dispatch(post, slot2, npad):
    mesh = plsc.VectorSubcoreMesh(core_axis_name="c", subcore_axis_name="s")
    return pl.kernel(
        _dispatch_body,
        out_type=jax.ShapeDtypeStruct((npad, _L), jnp.float32),
        mesh=mesh,
        scratch_types=[
            pltpu.VMEM((8, _L), jnp.int32),
            pltpu.VMEM((8, 1024), jnp.float32),
            pltpu.VMEM((512, _L), jnp.float32),
            pltpu.SemaphoreType.DMA,
        ],
        compiler_params=_SC_PARAMS,
    )(post, slot2)


def _mlp_body(e_ref, x_ref, w1_ref, b1_ref, w2_ref, b2_ref,
              w3_ref, b3_ref, w4_ref, b4_ref, y_ref):
    x = x_ref[...][:, :3]
    b = x.shape[0]
    h = jnp.tanh(jnp.dot(x.astype(jnp.bfloat16), w1_ref[0],
                         preferred_element_type=jnp.float32) + b1_ref[0])
    h = jnp.tanh(jnp.dot(h.astype(jnp.bfloat16), w2_ref[0],
                         preferred_element_type=jnp.float32) + b2_ref[0])
    h = jnp.tanh(jnp.dot(h.astype(jnp.bfloat16), w3_ref[0],
                         preferred_element_type=jnp.float32) + b3_ref[0])
    y = jnp.tanh(jnp.dot(h.astype(jnp.bfloat16), w4_ref[0],
                         preferred_element_type=jnp.float32) + b4_ref[0])
    y_ref[...] = jnp.concatenate(
        [y, jnp.zeros((b, _L - 3), jnp.float32)], axis=1)


def _mlp(block_expert, xpad, w1, b1, w2, b2, w3, b3, w4, b4):
    npad = xpad.shape[0]
    nb = npad // _B
    grid_spec = pltpu.PrefetchScalarGridSpec(
        num_scalar_prefetch=1,
        grid=(nb,),
        in_specs=[
            pl.BlockSpec((_B, _L), lambda i, e: (i, 0)),
            pl.BlockSpec((1, 3, _W), lambda i, e: (e[i, 0], 0, 0)),
            pl.BlockSpec((1, 1, _W), lambda i, e: (e[i, 0], 0, 0)),
            pl.BlockSpec((1, _W, _W), lambda i, e: (e[i, 0], 0, 0)),
            pl.BlockSpec((1, 1, _W), lambda i, e: (e[i, 0], 0, 0)),
            pl.BlockSpec((1, _W, _W), lambda i, e: (e[i, 0], 0, 0)),
            pl.BlockSpec((1, 1, _W), lambda i, e: (e[i, 0], 0, 0)),
            pl.BlockSpec((1, _W, 3), lambda i, e: (e[i, 0], 0, 0)),
            pl.BlockSpec((1, 1, 3), lambda i, e: (e[i, 0], 0, 0)),
        ],
        out_specs=pl.BlockSpec((_B, _L), lambda i, e: (i, 0)),
    )
    return pl.pallas_call(
        _mlp_body,
        grid_spec=grid_spec,
        out_shape=jax.ShapeDtypeStruct((npad, _L), jnp.float32),
    )(block_expert, xpad, w1.astype(jnp.bfloat16),
      b1.reshape(_C, 1, _W), w2.astype(jnp.bfloat16),
      b2.reshape(_C, 1, _W), w3.astype(jnp.bfloat16),
      b3.reshape(_C, 1, _W), w4.astype(jnp.bfloat16), b4.reshape(_C, 1, 3))


def _combine_body(ypad_hbm, slot2_hbm, out_hbm, idx2_v, y_v, sem):
    wid = lax.axis_index("s") * 2 + lax.axis_index("c")
    base = wid * 1024
    pltpu.sync_copy(slot2_hbm.at[pl.ds(wid * 8, 8)], idx2_v)
    for half in range(2):
        cps = [
            pltpu.async_copy(ypad_hbm.at[idx2_v.at[half * 4 + j]],
                             y_v.at[pl.ds(j * 128, 128)], sem)
            for j in range(4)
        ]
        for cp in cps:
            cp.wait()
        pltpu.sync_copy(y_v, out_hbm.at[pl.ds(base + half * 512, 512)])


def _combine(ypad, slot2):
    n = slot2.shape[0] * slot2.shape[1]
    mesh = plsc.VectorSubcoreMesh(core_axis_name="c", subcore_axis_name="s")
    return pl.kernel(
        _combine_body,
        out_type=jax.ShapeDtypeStruct((n, _L), jnp.float32),
        mesh=mesh,
        scratch_types=[
            pltpu.VMEM((8, _L), jnp.int32),
            pltpu.VMEM((512, _L), jnp.float32),
            pltpu.SemaphoreType.DMA,
        ],
        compiler_params=_SC_PARAMS,
    )(ypad, slot2)


def kernel(positions, times, W1, b1, W2, b2, W3, b3, W4, b4):
    n = positions.shape[0]
    npad = n + _C * _B
    nb = npad // _B

    times2 = times.astype(jnp.int32).reshape(n // _L, _L)
    slot2, block_expert = _route(times2, nb)
    post = jnp.pad(positions.T, ((0, 5), (0, 0)))
    xpad = _dispatch(post, slot2, npad)
    ypad = _mlp(block_expert, xpad, W1, b1, W2, b2, W3, b3, W4, b4)
    out = _combine(ypad, slot2)
    return out[:, :3]


# speedup vs baseline: 1.6770x; 1.1045x over previous
import jax
import jax.numpy as jnp
from jax import lax
from jax.experimental import pallas as pl
from jax.experimental.pallas import tpu as pltpu
from jax.experimental.pallas import tpu_sc as plsc

_C = 8
_W = 256
_B = 1024
_NW = 32
_L = 128

_SC_PARAMS = pltpu.CompilerParams(needs_layout_passes=False,
                                  use_tc_tiling_on_sc=True)


def _route_body(t_ref, slot_ref, be_ref):
    t = t_ref[...]
    rows = t.shape[0]
    nb = be_ref.shape[0]
    rank = jnp.zeros_like(t)
    ohs, totals = [None] * _C, [None] * _C
    for p in range(_C // 2):
        oh_lo = (t == p).astype(jnp.int32)
        oh_hi = (t == p + _C // 2).astype(jnp.int32)
        ohs[p], ohs[p + _C // 2] = oh_lo, oh_hi
        packed = oh_lo + (oh_hi << 16)
        pre = packed
        s = 1
        while s < _L:
            pre = pre + jnp.concatenate(
                [jnp.zeros((rows, s), jnp.int32), pre[:, : _L - s]], axis=1)
            s *= 2
        row_tot = pre[:, _L - 1 :]
        inc = row_tot
        s = 1
        while s < rows:
            inc = inc + jnp.concatenate(
                [jnp.zeros((s, 1), jnp.int32), inc[: rows - s, :]], axis=0)
            s *= 2
        row_off = inc - row_tot
        pre_full = pre + row_off
        lo = pre_full & 0xFFFF
        hi = (pre_full >> 16) & 0xFFFF
        rank = (rank + jnp.where(oh_lo == 1, lo - 1, 0)
                + jnp.where(oh_hi == 1, hi - 1, 0))
        tot = inc[rows - 1 :, :]
        totals[p] = tot & 0xFFFF
        totals[p + _C // 2] = (tot >> 16) & 0xFFFF

    run = jnp.zeros((1, 1), jnp.int32)
    slot = rank
    cum_pieces = []
    for c in range(_C):
        nblk_c = (totals[c] + (_B - 1)) // _B
        slot = slot + ohs[c] * (run * _B)
        run = run + nblk_c
        cum_pieces.append(run)
    slot_ref[...] = slot
    cum_row = jnp.concatenate(cum_pieces, axis=1)
    bid = lax.broadcasted_iota(jnp.int32, (nb, _C), 0)
    be = jnp.minimum(
        jnp.sum((bid >= cum_row).astype(jnp.int32), axis=1, keepdims=True),
        _C - 1)
    be_ref[...] = be


def _route(times2d, nb):
    return pl.pallas_call(
        _route_body,
        out_shape=(
            jax.ShapeDtypeStruct(times2d.shape, jnp.int32),
            jax.ShapeDtypeStruct((nb, 1), jnp.int32),
        ),
    )(times2d)


def _dispatch_body(post_hbm, slot2_hbm, xpad_hbm,
                   idx2_v, post_v, pos_v, sem):
    wid = lax.axis_index("s") * 2 + lax.axis_index("c")
    base = wid * 1024
    pltpu.sync_copy(slot2_hbm.at[pl.ds(wid * 8, 8)], idx2_v)
    pltpu.sync_copy(post_hbm.at[:, pl.ds(base, 1024)], post_v)

    lanes = lax.broadcasted_iota(jnp.int32, (16,), 0)
    for half in range(2):
        def body(j, carry, half=half):
            src = half * 512 + j * 16
            rows = lanes + j * 16
            for c in range(3):
                xv = post_v[c, pl.ds(src, 16)]
                plsc.store_scatter(pos_v, [rows, jnp.full((16,), c, jnp.int32)], xv)
            return carry
        lax.fori_loop(0, 32, body, 0)
        cps = [
            pltpu.async_copy(pos_v.at[pl.ds(j * 128, 128)],
                             xpad_hbm.at[idx2_v.at[half * 4 + j]], sem)
            for j in range(4)
        ]
        for cp in cps:
            cp.wait()


def _dispatch(post, slot2, npad):
    mesh = plsc.VectorSubcoreMesh(core_axis_name="c", subcore_axis_name="s")
    return pl.kernel(
        _dispatch_body,
        out_type=jax.ShapeDtypeStruct((npad, _L), jnp.float32),
        mesh=mesh,
        scratch_types=[
            pltpu.VMEM((8, _L), jnp.int32),
            pltpu.VMEM((8, 1024), jnp.float32),
            pltpu.VMEM((512, _L), jnp.float32),
            pltpu.SemaphoreType.DMA,
        ],
        compiler_params=_SC_PARAMS,
    )(post, slot2)


def _mlp_body(e_ref, x_ref, w1_ref, b1_ref, w2_ref, b2_ref,
              w3_ref, b3_ref, w4_ref, b4_ref, y_ref):
    x = x_ref[...][:, :3]
    b = x.shape[0]
    h = jnp.tanh(jnp.dot(x.astype(jnp.bfloat16), w1_ref[0],
                         preferred_element_type=jnp.float32) + b1_ref[0])
    h = jnp.tanh(jnp.dot(h.astype(jnp.bfloat16), w2_ref[0],
                         preferred_element_type=jnp.float32) + b2_ref[0])
    h = jnp.tanh(jnp.dot(h.astype(jnp.bfloat16), w3_ref[0],
                         preferred_element_type=jnp.float32) + b3_ref[0])
    y = jnp.tanh(jnp.dot(h.astype(jnp.bfloat16), w4_ref[0],
                         preferred_element_type=jnp.float32) + b4_ref[0])
    y_ref[...] = jnp.concatenate(
        [y, jnp.zeros((b, _L - 3), jnp.float32)], axis=1)


def _mlp(block_expert, xpad, w1, b1, w2, b2, w3, b3, w4, b4):
    npad = xpad.shape[0]
    nb = npad // _B
    grid_spec = pltpu.PrefetchScalarGridSpec(
        num_scalar_prefetch=1,
        grid=(nb,),
        in_specs=[
            pl.BlockSpec((_B, _L), lambda i, e: (i, 0)),
            pl.BlockSpec((1, 3, _W), lambda i, e: (e[i, 0], 0, 0)),
            pl.BlockSpec((1, 1, _W), lambda i, e: (e[i, 0], 0, 0)),
            pl.BlockSpec((1, _W, _W), lambda i, e: (e[i, 0], 0, 0)),
            pl.BlockSpec((1, 1, _W), lambda i, e: (e[i, 0], 0, 0)),
            pl.BlockSpec((1, _W, _W), lambda i, e: (e[i, 0], 0, 0)),
            pl.BlockSpec((1, 1, _W), lambda i, e: (e[i, 0], 0, 0)),
            pl.BlockSpec((1, _W, 3), lambda i, e: (e[i, 0], 0, 0)),
            pl.BlockSpec((1, 1, 3), lambda i, e: (e[i, 0], 0, 0)),
        ],
        out_specs=pl.BlockSpec((_B, _L), lambda i, e: (i, 0)),
    )
    return pl.pallas_call(
        _mlp_body,
        grid_spec=grid_spec,
        out_shape=jax.ShapeDtypeStruct((npad, _L), jnp.float32),
    )(block_expert, xpad, w1.astype(jnp.bfloat16),
      b1.reshape(_C, 1, _W), w2.astype(jnp.bfloat16),
      b2.reshape(_C, 1, _W), w3.astype(jnp.bfloat16),
      b3.reshape(_C, 1, _W), w4.astype(jnp.bfloat16), b4.reshape(_C, 1, 3))


def _combine_body(ypad_hbm, slot2_hbm, out_hbm, idx2_v, y_v, y8_v, sem):
    wid = lax.axis_index("s") * 2 + lax.axis_index("c")
    base = wid * 1024
    pltpu.sync_copy(slot2_hbm.at[pl.ds(wid * 8, 8)], idx2_v)
    lanes = lax.broadcasted_iota(jnp.int32, (16,), 0)
    for half in range(2):
        cps = [
            pltpu.async_copy(ypad_hbm.at[idx2_v.at[half * 4 + j]],
                             y_v.at[pl.ds(j * 128, 128)], sem)
            for j in range(4)
        ]
        for cp in cps:
            cp.wait()

        def body(j, carry, half=half):
            rows = lanes + j * 16
            for c in range(3):
                vals = plsc.load_gather(
                    y_v, [rows, jnp.full((16,), c, jnp.int32)])
                y8_v[c, pl.ds(half * 512 + j * 16, 16)] = vals
            return carry
        lax.fori_loop(0, 32, body, 0)
    pltpu.sync_copy(y8_v, out_hbm.at[:, pl.ds(base, 1024)])


def _combine(ypad, slot2):
    n = slot2.shape[0] * slot2.shape[1]
    mesh = plsc.VectorSubcoreMesh(core_axis_name="c", subcore_axis_name="s")
    return pl.kernel(
        _combine_body,
        out_type=jax.ShapeDtypeStruct((8, n), jnp.float32),
        mesh=mesh,
        scratch_types=[
            pltpu.VMEM((8, _L), jnp.int32),
            pltpu.VMEM((512, _L), jnp.float32),
            pltpu.VMEM((8, 1024), jnp.float32),
            pltpu.SemaphoreType.DMA,
        ],
        compiler_params=_SC_PARAMS,
    )(ypad, slot2)


def kernel(positions, times, W1, b1, W2, b2, W3, b3, W4, b4):
    n = positions.shape[0]
    npad = n + _C * _B
    nb = npad // _B

    times2 = times.astype(jnp.int32).reshape(n // _L, _L)
    slot2, block_expert = _route(times2, nb)
    post = jnp.pad(positions.T, ((0, 5), (0, 0)))
    xpad = _dispatch(post, slot2, npad)
    ypad = _mlp(block_expert, xpad, W1, b1, W2, b2, W3, b3, W4, b4)
    out8 = _combine(ypad, slot2)
    return out8[:3, :].T


# speedup vs baseline: 1.6806x; 1.0021x over previous
import jax
import jax.numpy as jnp
from jax import lax
from jax.experimental import pallas as pl
from jax.experimental.pallas import tpu as pltpu
from jax.experimental.pallas import tpu_sc as plsc

_C = 8
_W = 256
_B = 1024
_NW = 32
_L = 128

_SC_PARAMS = pltpu.CompilerParams(needs_layout_passes=False,
                                  use_tc_tiling_on_sc=True)


def _route_body(t_ref, slot_ref, be_ref):
    t = t_ref[...]
    rows = t.shape[0]
    nb = be_ref.shape[0]
    rank = jnp.zeros_like(t)
    ohs, totals = [None] * _C, [None] * _C
    for p in range(_C // 2):
        oh_lo = (t == p).astype(jnp.int32)
        oh_hi = (t == p + _C // 2).astype(jnp.int32)
        ohs[p], ohs[p + _C // 2] = oh_lo, oh_hi
        packed = oh_lo + (oh_hi << 16)
        pre = packed
        s = 1
        while s < _L:
            pre = pre + jnp.concatenate(
                [jnp.zeros((rows, s), jnp.int32), pre[:, : _L - s]], axis=1)
            s *= 2
        row_tot = pre[:, _L - 1 :]
        inc = row_tot
        s = 1
        while s < rows:
            inc = inc + jnp.concatenate(
                [jnp.zeros((s, 1), jnp.int32), inc[: rows - s, :]], axis=0)
            s *= 2
        row_off = inc - row_tot
        pre_full = pre + row_off
        lo = pre_full & 0xFFFF
        hi = (pre_full >> 16) & 0xFFFF
        rank = (rank + jnp.where(oh_lo == 1, lo - 1, 0)
                + jnp.where(oh_hi == 1, hi - 1, 0))
        tot = inc[rows - 1 :, :]
        totals[p] = tot & 0xFFFF
        totals[p + _C // 2] = (tot >> 16) & 0xFFFF

    run = jnp.zeros((1, 1), jnp.int32)
    slot = rank
    cum_pieces = []
    for c in range(_C):
        nblk_c = (totals[c] + (_B - 1)) // _B
        slot = slot + ohs[c] * (run * _B)
        run = run + nblk_c
        cum_pieces.append(run)
    slot_ref[...] = slot
    cum_row = jnp.concatenate(cum_pieces, axis=1)
    bid = lax.broadcasted_iota(jnp.int32, (nb, _C), 0)
    be = jnp.minimum(
        jnp.sum((bid >= cum_row).astype(jnp.int32), axis=1, keepdims=True),
        _C - 1)
    be_ref[...] = be


def _route(times2d, nb):
    return pl.pallas_call(
        _route_body,
        out_shape=(
            jax.ShapeDtypeStruct(times2d.shape, jnp.int32),
            jax.ShapeDtypeStruct((nb, 1), jnp.int32),
        ),
    )(times2d)


def _dispatch_body(post_hbm, slot2_hbm, xpad_hbm,
                   idx2_v, post_v, pos_v, sem):
    wid = lax.axis_index("s") * 2 + lax.axis_index("c")
    base = wid * 1024
    pltpu.sync_copy(slot2_hbm.at[pl.ds(wid * 8, 8)], idx2_v)
    pltpu.sync_copy(post_hbm.at[:, pl.ds(base, 1024)], post_v)

    lanes = lax.broadcasted_iota(jnp.int32, (16,), 0)
    for half in range(2):
        def body(j, carry, half=half):
            src = half * 512 + j * 16
            rows = lanes + j * 16
            for c in range(3):
                xv = post_v[c, pl.ds(src, 16)]
                plsc.store_scatter(pos_v, [rows, jnp.full((16,), c, jnp.int32)], xv)
            return carry
        lax.fori_loop(0, 32, body, 0)
        cps = [
            pltpu.async_copy(pos_v.at[pl.ds(j * 128, 128)],
                             xpad_hbm.at[idx2_v.at[half * 4 + j]], sem)
            for j in range(4)
        ]
        for cp in cps:
            cp.wait()


def _dispatch(post, slot2, npad):
    mesh = plsc.VectorSubcoreMesh(core_axis_name="c", subcore_axis_name="s")
    return pl.kernel(
        _dispatch_body,
        out_type=jax.ShapeDtypeStruct((npad, _L), jnp.float32),
        mesh=mesh,
        scratch_types=[
            pltpu.VMEM((8, _L), jnp.int32),
            pltpu.VMEM((8, 1024), jnp.float32),
            pltpu.VMEM((512, _L), jnp.float32),
            pltpu.SemaphoreType.DMA,
        ],
        compiler_params=_SC_PARAMS,
    )(post, slot2)


def _mlp_body(e_ref, x_ref, w1_ref, b1_ref, w2_ref, b2_ref,
              w3_ref, b3_ref, w4_ref, b4_ref, y_ref):
    x = x_ref[:, :3]
    b = x.shape[0]
    h = jnp.tanh(jnp.dot(x.astype(jnp.bfloat16), w1_ref[0],
                         preferred_element_type=jnp.float32) + b1_ref[0])
    h = jnp.tanh(jnp.dot(h.astype(jnp.bfloat16), w2_ref[0],
                         preferred_element_type=jnp.float32) + b2_ref[0])
    h = jnp.tanh(jnp.dot(h.astype(jnp.bfloat16), w3_ref[0],
                         preferred_element_type=jnp.float32) + b3_ref[0])
    y = jnp.tanh(jnp.dot(h.astype(jnp.bfloat16), w4_ref[0],
                         preferred_element_type=jnp.float32) + b4_ref[0])
    y_ref[:, :3] = y


def _mlp(block_expert, xpad, w1, b1, w2, b2, w3, b3, w4, b4):
    npad = xpad.shape[0]
    nb = npad // _B
    grid_spec = pltpu.PrefetchScalarGridSpec(
        num_scalar_prefetch=1,
        grid=(nb,),
        in_specs=[
            pl.BlockSpec((_B, _L), lambda i, e: (i, 0)),
            pl.BlockSpec((1, 3, _W), lambda i, e: (e[i, 0], 0, 0)),
            pl.BlockSpec((1, 1, _W), lambda i, e: (e[i, 0], 0, 0)),
            pl.BlockSpec((1, _W, _W), lambda i, e: (e[i, 0], 0, 0)),
            pl.BlockSpec((1, 1, _W), lambda i, e: (e[i, 0], 0, 0)),
            pl.BlockSpec((1, _W, _W), lambda i, e: (e[i, 0], 0, 0)),
            pl.BlockSpec((1, 1, _W), lambda i, e: (e[i, 0], 0, 0)),
            pl.BlockSpec((1, _W, 3), lambda i, e: (e[i, 0], 0, 0)),
            pl.BlockSpec((1, 1, 3), lambda i, e: (e[i, 0], 0, 0)),
        ],
        out_specs=pl.BlockSpec((_B, _L), lambda i, e: (i, 0)),
    )
    return pl.pallas_call(
        _mlp_body,
        grid_spec=grid_spec,
        out_shape=jax.ShapeDtypeStruct((npad, _L), jnp.float32),
    )(block_expert, xpad, w1.astype(jnp.bfloat16),
      b1.reshape(_C, 1, _W), w2.astype(jnp.bfloat16),
      b2.reshape(_C, 1, _W), w3.astype(jnp.bfloat16),
      b3.reshape(_C, 1, _W), w4.astype(jnp.bfloat16), b4.reshape(_C, 1, 3))


def _combine_body(ypad_hbm, slot2_hbm, out_hbm, idx2_v, y_v, y8_v, sem):
    wid = lax.axis_index("s") * 2 + lax.axis_index("c")
    base = wid * 1024
    pltpu.sync_copy(slot2_hbm.at[pl.ds(wid * 8, 8)], idx2_v)
    lanes = lax.broadcasted_iota(jnp.int32, (16,), 0)
    for half in range(2):
        cps = [
            pltpu.async_copy(ypad_hbm.at[idx2_v.at[half * 4 + j]],
                             y_v.at[pl.ds(j * 128, 128)], sem)
            for j in range(4)
        ]
        for cp in cps:
            cp.wait()

        def body(j, carry, half=half):
            rows = lanes + j * 16
            for c in range(3):
                vals = plsc.load_gather(
                    y_v, [rows, jnp.full((16,), c, jnp.int32)])
                y8_v[c, pl.ds(half * 512 + j * 16, 16)] = vals
            return carry
        lax.fori_loop(0, 32, body, 0)
    pltpu.sync_copy(y8_v, out_hbm.at[:, pl.ds(base, 1024)])


def _combine(ypad, slot2):
    n = slot2.shape[0] * slot2.shape[1]
    mesh = plsc.VectorSubcoreMesh(core_axis_name="c", subcore_axis_name="s")
    return pl.kernel(
        _combine_body,
        out_type=jax.ShapeDtypeStruct((8, n), jnp.float32),
        mesh=mesh,
        scratch_types=[
            pltpu.VMEM((8, _L), jnp.int32),
            pltpu.VMEM((512, _L), jnp.float32),
            pltpu.VMEM((8, 1024), jnp.float32),
            pltpu.SemaphoreType.DMA,
        ],
        compiler_params=_SC_PARAMS,
    )(ypad, slot2)


def kernel(positions, times, W1, b1, W2, b2, W3, b3, W4, b4):
    n = positions.shape[0]
    npad = n + _C * _B
    nb = npad // _B

    times2 = times.astype(jnp.int32).reshape(n // _L, _L)
    slot2, block_expert = _route(times2, nb)
    post = jnp.pad(positions.T, ((0, 5), (0, 0)))
    xpad = _dispatch(post, slot2, npad)
    ypad = _mlp(block_expert, xpad, W1, b1, W2, b2, W3, b3, W4, b4)
    out8 = _combine(ypad, slot2)
    return out8[:3, :].T


# speedup vs baseline: 1.7723x; 1.0546x over previous
import jax
import jax.numpy as jnp
from jax import lax
from jax.experimental import pallas as pl
from jax.experimental.pallas import tpu as pltpu
from jax.experimental.pallas import tpu_sc as plsc

_C = 8
_W = 256
_B = 2048
_NW = 32
_L = 128

_SC_PARAMS = pltpu.CompilerParams(needs_layout_passes=False,
                                  use_tc_tiling_on_sc=True)


def _route_body(t_ref, slot_ref, be_ref):
    t = t_ref[...]
    rows = t.shape[0]
    nb = be_ref.shape[0]
    rank = jnp.zeros_like(t)
    ohs, totals = [None] * _C, [None] * _C
    for p in range(_C // 2):
        oh_lo = (t == p).astype(jnp.int32)
        oh_hi = (t == p + _C // 2).astype(jnp.int32)
        ohs[p], ohs[p + _C // 2] = oh_lo, oh_hi
        packed = oh_lo + (oh_hi << 16)
        pre = packed
        s = 1
        while s < _L:
            pre = pre + jnp.concatenate(
                [jnp.zeros((rows, s), jnp.int32), pre[:, : _L - s]], axis=1)
            s *= 2
        row_tot = pre[:, _L - 1 :]
        inc = row_tot
        s = 1
        while s < rows:
            inc = inc + jnp.concatenate(
                [jnp.zeros((s, 1), jnp.int32), inc[: rows - s, :]], axis=0)
            s *= 2
        row_off = inc - row_tot
        pre_full = pre + row_off
        lo = pre_full & 0xFFFF
        hi = (pre_full >> 16) & 0xFFFF
        rank = (rank + jnp.where(oh_lo == 1, lo - 1, 0)
                + jnp.where(oh_hi == 1, hi - 1, 0))
        tot = inc[rows - 1 :, :]
        totals[p] = tot & 0xFFFF
        totals[p + _C // 2] = (tot >> 16) & 0xFFFF

    run = jnp.zeros((1, 1), jnp.int32)
    slot = rank
    cum_pieces = []
    for c in range(_C):
        nblk_c = (totals[c] + (_B - 1)) // _B
        slot = slot + ohs[c] * (run * _B)
        run = run + nblk_c
        cum_pieces.append(run)
    slot_ref[...] = slot
    cum_row = jnp.concatenate(cum_pieces, axis=1)
    bid = lax.broadcasted_iota(jnp.int32, (nb, _C), 0)
    be = jnp.minimum(
        jnp.sum((bid >= cum_row).astype(jnp.int32), axis=1, keepdims=True),
        _C - 1)
    be_ref[...] = be


def _route(times2d, nb):
    return pl.pallas_call(
        _route_body,
        out_shape=(
            jax.ShapeDtypeStruct(times2d.shape, jnp.int32),
            jax.ShapeDtypeStruct((nb, 1), jnp.int32),
        ),
    )(times2d)


def _dispatch_body(post_hbm, slot2_hbm, xpad_hbm,
                   idx2_v, post_v, pos_v, sem):
    wid = lax.axis_index("s") * 2 + lax.axis_index("c")
    base = wid * 1024
    pltpu.sync_copy(slot2_hbm.at[pl.ds(wid * 8, 8)], idx2_v)
    pltpu.sync_copy(post_hbm.at[:, pl.ds(base, 1024)], post_v)

    lanes = lax.broadcasted_iota(jnp.int32, (16,), 0)
    for half in range(2):
        def body(j, carry, half=half):
            src = half * 512 + j * 16
            rows = lanes + j * 16
            for c in range(3):
                xv = post_v[c, pl.ds(src, 16)]
                plsc.store_scatter(pos_v, [rows, jnp.full((16,), c, jnp.int32)], xv)
            return carry
        lax.fori_loop(0, 32, body, 0)
        cps = [
            pltpu.async_copy(pos_v.at[pl.ds(j * 128, 128)],
                             xpad_hbm.at[idx2_v.at[half * 4 + j]], sem)
            for j in range(4)
        ]
        for cp in cps:
            cp.wait()


def _dispatch(post, slot2, npad):
    mesh = plsc.VectorSubcoreMesh(core_axis_name="c", subcore_axis_name="s")
    return pl.kernel(
        _dispatch_body,
        out_type=jax.ShapeDtypeStruct((npad, _L), jnp.float32),
        mesh=mesh,
        scratch_types=[
            pltpu.VMEM((8, _L), jnp.int32),
            pltpu.VMEM((8, 1024), jnp.float32),
            pltpu.VMEM((512, _L), jnp.float32),
            pltpu.SemaphoreType.DMA,
        ],
        compiler_params=_SC_PARAMS,
    )(post, slot2)


def _mlp_body(e_ref, x_ref, w1_ref, b1_ref, w2_ref, b2_ref,
              w3_ref, b3_ref, w4_ref, b4_ref, y_ref):
    x = x_ref[:, :3]
    b = x.shape[0]
    h = jnp.tanh(jnp.dot(x.astype(jnp.bfloat16), w1_ref[0],
                         preferred_element_type=jnp.float32) + b1_ref[0])
    h = jnp.tanh(jnp.dot(h.astype(jnp.bfloat16), w2_ref[0],
                         preferred_element_type=jnp.float32) + b2_ref[0])
    h = jnp.tanh(jnp.dot(h.astype(jnp.bfloat16), w3_ref[0],
                         preferred_element_type=jnp.float32) + b3_ref[0])
    y = jnp.tanh(jnp.dot(h.astype(jnp.bfloat16), w4_ref[0],
                         preferred_element_type=jnp.float32) + b4_ref[0])
    y_ref[:, :3] = y


def _mlp(block_expert, xpad, w1, b1, w2, b2, w3, b3, w4, b4):
    npad = xpad.shape[0]
    nb = npad // _B
    grid_spec = pltpu.PrefetchScalarGridSpec(
        num_scalar_prefetch=1,
        grid=(nb,),
        in_specs=[
            pl.BlockSpec((_B, _L), lambda i, e: (i, 0)),
            pl.BlockSpec((1, 3, _W), lambda i, e: (e[i, 0], 0, 0)),
            pl.BlockSpec((1, 1, _W), lambda i, e: (e[i, 0], 0, 0)),
            pl.BlockSpec((1, _W, _W), lambda i, e: (e[i, 0], 0, 0)),
            pl.BlockSpec((1, 1, _W), lambda i, e: (e[i, 0], 0, 0)),
            pl.BlockSpec((1, _W, _W), lambda i, e: (e[i, 0], 0, 0)),
            pl.BlockSpec((1, 1, _W), lambda i, e: (e[i, 0], 0, 0)),
            pl.BlockSpec((1, _W, 3), lambda i, e: (e[i, 0], 0, 0)),
            pl.BlockSpec((1, 1, 3), lambda i, e: (e[i, 0], 0, 0)),
        ],
        out_specs=pl.BlockSpec((_B, _L), lambda i, e: (i, 0)),
    )
    return pl.pallas_call(
        _mlp_body,
        grid_spec=grid_spec,
        out_shape=jax.ShapeDtypeStruct((npad, _L), jnp.float32),
    )(block_expert, xpad, w1.astype(jnp.bfloat16),
      b1.reshape(_C, 1, _W), w2.astype(jnp.bfloat16),
      b2.reshape(_C, 1, _W), w3.astype(jnp.bfloat16),
      b3.reshape(_C, 1, _W), w4.astype(jnp.bfloat16), b4.reshape(_C, 1, 3))


def _combine_body(ypad_hbm, slot2_hbm, out_hbm, idx2_v, y_v, y8_v, sem):
    wid = lax.axis_index("s") * 2 + lax.axis_index("c")
    base = wid * 1024
    pltpu.sync_copy(slot2_hbm.at[pl.ds(wid * 8, 8)], idx2_v)
    lanes = lax.broadcasted_iota(jnp.int32, (16,), 0)
    for half in range(2):
        cps = [
            pltpu.async_copy(ypad_hbm.at[idx2_v.at[half * 4 + j]],
                             y_v.at[pl.ds(j * 128, 128)], sem)
            for j in range(4)
        ]
        for cp in cps:
            cp.wait()

        def body(j, carry, half=half):
            rows = lanes + j * 16
            for c in range(3):
                vals = plsc.load_gather(
                    y_v, [rows, jnp.full((16,), c, jnp.int32)])
                y8_v[c, pl.ds(half * 512 + j * 16, 16)] = vals
            return carry
        lax.fori_loop(0, 32, body, 0)
    pltpu.sync_copy(y8_v, out_hbm.at[:, pl.ds(base, 1024)])


def _combine(ypad, slot2):
    n = slot2.shape[0] * slot2.shape[1]
    mesh = plsc.VectorSubcoreMesh(core_axis_name="c", subcore_axis_name="s")
    return pl.kernel(
        _combine_body,
        out_type=jax.ShapeDtypeStruct((8, n), jnp.float32),
        mesh=mesh,
        scratch_types=[
            pltpu.VMEM((8, _L), jnp.int32),
            pltpu.VMEM((512, _L), jnp.float32),
            pltpu.VMEM((8, 1024), jnp.float32),
            pltpu.SemaphoreType.DMA,
        ],
        compiler_params=_SC_PARAMS,
    )(ypad, slot2)


def kernel(positions, times, W1, b1, W2, b2, W3, b3, W4, b4):
    n = positions.shape[0]
    npad = n + _C * _B
    nb = npad // _B

    times2 = times.astype(jnp.int32).reshape(n // _L, _L)
    slot2, block_expert = _route(times2, nb)
    post = jnp.pad(positions.T, ((0, 5), (0, 0)))
    xpad = _dispatch(post, slot2, npad)
    ypad = _mlp(block_expert, xpad, W1, b1, W2, b2, W3, b3, W4, b4)
    out8 = _combine(ypad, slot2)
    return out8[:3, :].T
